# baseline (speedup 1.0000x reference)
import functools

import jax
import jax.numpy as jnp
from jax import lax
from jax.experimental import pallas as pl
from jax.experimental.pallas import tpu as pltpu
from jax.experimental.pallas import tpu_sc as plsc

NUM_CLASSES = 1000
BATCH = 1024
SEQ = 26
N_ROWS = BATCH * SEQ
NUM_CORES = 2
NUM_SUBCORES = 16
NW = NUM_CORES * NUM_SUBCORES
ROWS_PER_W = N_ROWS // NW
CHUNK = 64
NCHUNK = ROWS_PER_W // CHUNK
BUF_WORDS = CHUNK * NUM_CLASSES
GROUPS = CHUNK // 16

_mesh = plsc.VectorSubcoreMesh(core_axis_name="c", subcore_axis_name="s")


@functools.partial(
    pl.kernel,
    out_type=jax.ShapeDtypeStruct((N_ROWS * NUM_CLASSES,), jnp.float32),
    mesh=_mesh,
    scratch_types=[
        pltpu.VMEM((ROWS_PER_W,), jnp.int32),
        pltpu.VMEM((BUF_WORDS,), jnp.float32),
        pltpu.VMEM((BUF_WORDS,), jnp.float32),
        pltpu.SemaphoreType.DMA,
        pltpu.SemaphoreType.DMA,
    ],
    compiler_params=pltpu.CompilerParams(needs_layout_passes=False),
)
def _onehot_sc(idx_hbm, zeros_hbm, out_hbm, idx_v, buf0, buf1, sem0, sem1):
    wid = lax.axis_index("s") * NUM_CORES + lax.axis_index("c")
    row0 = wid * ROWS_PER_W

    z0 = pltpu.async_copy(zeros_hbm, buf0, sem0)
    z1 = pltpu.async_copy(zeros_hbm, buf1, sem1)
    pltpu.sync_copy(idx_hbm.at[pl.ds(row0, ROWS_PER_W)], idx_v)
    z0.wait()
    z1.wait()

    bufs = (buf0, buf1)
    sems = (sem0, sem1)
    lane = lax.iota(jnp.int32, 16)
    ones16 = jnp.full((16,), 1.0, jnp.float32)
    zeros16 = jnp.zeros((16,), jnp.float32)

    def chunk_offsets(c, g):
        idx16 = idx_v[pl.ds(c * CHUNK + g * 16, 16)]
        return (g * 16 + lane) * NUM_CLASSES + idx16

    pending = [None, None]
    pending_chunk = [0, 0]
    for c in range(NCHUNK):
        b = c % 2
        if pending[b] is not None:
            pending[b].wait()
            for g in range(GROUPS):
                plsc.store_scatter(
                    bufs[b], [chunk_offsets(pending_chunk[b], g)], zeros16)
        for g in range(GROUPS):
            plsc.store_scatter(bufs[b], [chunk_offsets(c, g)], ones16)
        dst = out_hbm.at[pl.ds(row0 * NUM_CLASSES + c * BUF_WORDS, BUF_WORDS)]
        pending[b] = pltpu.async_copy(bufs[b], dst, sems[b])
        pending_chunk[b] = c
    for b in (0, 1):
        if pending[b] is not None:
            pending[b].wait()


def kernel(x):
    idx = x.reshape(N_ROWS).astype(jnp.int32)
    zeros = jnp.zeros((BUF_WORDS,), jnp.float32)
    y = _onehot_sc(idx, zeros)
    return y.reshape(BATCH, SEQ, NUM_CLASSES)


# speedup vs baseline: 1.0079x; 1.0079x over previous
import functools

import jax
import jax.numpy as jnp
import numpy as np
from jax import lax
from jax.experimental import pallas as pl
from jax.experimental.pallas import tpu as pltpu
from jax.experimental.pallas import tpu_sc as plsc

NUM_CLASSES = 1000
BATCH = 1024
SEQ = 26
NUM_CORES = 2
NUM_SUBCORES = 16
NW = NUM_CORES * NUM_SUBCORES
BROWS_PER_W = BATCH // NW
CHUNK_B = 2
NCHUNK = BROWS_PER_W // CHUNK_B
FLAT_PER_CHUNK = CHUNK_B * SEQ
GROUPS = 4

_mesh = plsc.VectorSubcoreMesh(core_axis_name="c", subcore_axis_name="s")


@functools.partial(
    pl.kernel,
    out_type=jax.ShapeDtypeStruct((BATCH, SEQ, NUM_CLASSES), jnp.float32),
    mesh=_mesh,
    scratch_types=[
        pltpu.VMEM((BROWS_PER_W * SEQ,), jnp.int32),
        pltpu.VMEM((CHUNK_B, SEQ, NUM_CLASSES), jnp.float32),
        pltpu.VMEM((CHUNK_B, SEQ, NUM_CLASSES), jnp.float32),
        pltpu.SemaphoreType.DMA,
        pltpu.SemaphoreType.DMA,
    ],
    compiler_params=pltpu.CompilerParams(
        needs_layout_passes=False, use_tc_tiling_on_sc=False),
)
def _onehot_sc(idx_hbm, zeros_hbm, out_hbm, idx_v, buf0, buf1, sem0, sem1):
    wid = lax.axis_index("s") * NUM_CORES + lax.axis_index("c")
    brow0 = wid * BROWS_PER_W

    z0 = pltpu.async_copy(zeros_hbm, buf0, sem0)
    z1 = pltpu.async_copy(zeros_hbm, buf1, sem1)
    pltpu.sync_copy(idx_hbm.at[pl.ds(brow0 * SEQ, BROWS_PER_W * SEQ)], idx_v)
    z0.wait()
    z1.wait()

    bufs = (buf0, buf1)
    sems = (sem0, sem1)
    ones16 = jnp.full((16,), 1.0, jnp.float32)
    zeros16 = jnp.zeros((16,), jnp.float32)

    lane = lax.iota(jnp.int32, 16)
    laneoff = []
    r_const = []
    s_const = []
    for g in range(GROUPS):
        lf = jnp.minimum(lane + g * 16, FLAT_PER_CHUNK - 1)
        r = lf // SEQ
        laneoff.append(lf)
        r_const.append(r)
        s_const.append(lf - r * SEQ)

    pending = [None, None]
    pending_chunk = [0, 0]
    for c in range(NCHUNK):
        b = c % 2
        if pending[b] is not None:
            pending[b].wait()
            pc = pending_chunk[b]
            for g in range(GROUPS):
                kv = plsc.load_gather(
                    idx_v, [pc * FLAT_PER_CHUNK + laneoff[g]])
                plsc.store_scatter(
                    bufs[b], [r_const[g], s_const[g], kv], zeros16)
        for g in range(GROUPS):
            kv = plsc.load_gather(idx_v, [c * FLAT_PER_CHUNK + laneoff[g]])
            plsc.store_scatter(
                bufs[b], [r_const[g], s_const[g], kv], ones16)
        dst = out_hbm.at[pl.ds(brow0 + c * CHUNK_B, CHUNK_B)]
        pending[b] = pltpu.async_copy(bufs[b], dst, sems[b])
        pending_chunk[b] = c
    for b in (0, 1):
        if pending[b] is not None:
            pending[b].wait()


def kernel(x):
    idx = x.reshape(BATCH * SEQ).astype(jnp.int32)
    zeros = jnp.zeros((CHUNK_B, SEQ, NUM_CLASSES), jnp.float32)
    return _onehot_sc(idx, zeros)


# speedup vs baseline: 1.8082x; 1.7940x over previous
import functools

import jax
import jax.numpy as jnp
import numpy as np
from jax import lax
from jax.experimental import pallas as pl
from jax.experimental.pallas import tpu as pltpu
from jax.experimental.pallas import tpu_sc as plsc

NUM_CLASSES = 1000
BATCH = 1024
SEQ = 26
NUM_CORES = 2
NUM_SUBCORES = 16
NW = NUM_CORES * NUM_SUBCORES
BROWS_PER_W = BATCH // NW
CHUNK_B = 1
NCHUNK = BROWS_PER_W // CHUNK_B
FLAT_PER_CHUNK = CHUNK_B * SEQ
GROUPS = 2

_mesh = plsc.VectorSubcoreMesh(core_axis_name="c", subcore_axis_name="s")


@functools.partial(
    pl.kernel,
    out_type=jax.ShapeDtypeStruct((BATCH, SEQ, NUM_CLASSES), jnp.float32),
    mesh=_mesh,
    scratch_types=[
        pltpu.VMEM((BROWS_PER_W * SEQ,), jnp.int32),
        pltpu.VMEM((CHUNK_B, SEQ, NUM_CLASSES), jnp.float32),
        pltpu.VMEM((CHUNK_B, SEQ, NUM_CLASSES), jnp.float32),
        pltpu.SemaphoreType.DMA,
        pltpu.SemaphoreType.DMA,
    ],
    compiler_params=pltpu.CompilerParams(needs_layout_passes=False),
)
def _onehot_sc(idx_hbm, zeros_hbm, out_hbm, idx_v, buf0, buf1, sem0, sem1):
    wid = lax.axis_index("s") * NUM_CORES + lax.axis_index("c")
    brow0 = wid * BROWS_PER_W

    z0 = pltpu.async_copy(zeros_hbm, buf0, sem0)
    z1 = pltpu.async_copy(zeros_hbm, buf1, sem1)
    pltpu.sync_copy(idx_hbm.at[pl.ds(brow0 * SEQ, BROWS_PER_W * SEQ)], idx_v)
    z0.wait()
    z1.wait()

    bufs = (buf0, buf1)
    sems = (sem0, sem1)
    ones16 = jnp.full((16,), 1.0, jnp.float32)
    zeros16 = jnp.zeros((16,), jnp.float32)

    lane = lax.iota(jnp.int32, 16)
    laneoff = []
    r_const = []
    s_const = []
    for g in range(GROUPS):
        lf = jnp.minimum(lane + g * 16, FLAT_PER_CHUNK - 1)
        r = lf // SEQ
        laneoff.append(lf)
        r_const.append(r)
        s_const.append(lf - r * SEQ)

    pending = [None, None]
    pending_chunk = [0, 0]
    for c in range(NCHUNK):
        b = c % 2
        if pending[b] is not None:
            pending[b].wait()
            pc = pending_chunk[b]
            for g in range(GROUPS):
                kv = plsc.load_gather(
                    idx_v, [pc * FLAT_PER_CHUNK + laneoff[g]])
                plsc.store_scatter(
                    bufs[b], [r_const[g], s_const[g], kv], zeros16)
        for g in range(GROUPS):
            kv = plsc.load_gather(idx_v, [c * FLAT_PER_CHUNK + laneoff[g]])
            plsc.store_scatter(
                bufs[b], [r_const[g], s_const[g], kv], ones16)
        dst = out_hbm.at[pl.ds(brow0 + c * CHUNK_B, CHUNK_B)]
        pending[b] = pltpu.async_copy(bufs[b], dst, sems[b])
        pending_chunk[b] = c
    for b in (0, 1):
        if pending[b] is not None:
            pending[b].wait()


def kernel(x):
    idx = x.reshape(BATCH * SEQ).astype(jnp.int32)
    zeros = jnp.zeros((CHUNK_B, SEQ, NUM_CLASSES), jnp.float32)
    return _onehot_sc(idx, zeros)


# speedup vs baseline: 4.9811x; 2.7547x over previous
import functools

import jax
import jax.numpy as jnp
from jax import lax
from jax.experimental import pallas as pl
from jax.experimental.pallas import tpu as pltpu
from jax.experimental.pallas import tpu_sc as plsc

NUM_CLASSES = 1000
BATCH = 1024
SEQ = 26
NUM_CORES = 2
NUM_SUBCORES = 16
KCH = 40
NKC = NUM_CLASSES // KCH
GROUPS = BATCH // 16
KDIV_MAGIC = 1639

_mesh = plsc.VectorSubcoreMesh(core_axis_name="c", subcore_axis_name="s")


@functools.partial(
    pl.kernel,
    out_type=jax.ShapeDtypeStruct((SEQ, NUM_CLASSES, BATCH), jnp.float32),
    mesh=_mesh,
    scratch_types=[
        pltpu.VMEM((1, KCH, BATCH), jnp.float32),
        pltpu.VMEM((1, KCH, BATCH), jnp.float32),
        pltpu.VMEM((BATCH,), jnp.int32),
        pltpu.VMEM((BATCH,), jnp.int32),
        pltpu.VMEM((BATCH,), jnp.int32),
        pltpu.SemaphoreType.DMA,
        pltpu.SemaphoreType.DMA,
    ],
    compiler_params=pltpu.CompilerParams(needs_layout_passes=False),
)
def _onehot_sc(xt_hbm, zeros_hbm, out_hbm,
               buf0, buf1, col, kcv, relv, sem0, sem1):
    wid = lax.axis_index("s") * NUM_CORES + lax.axis_index("c")

    @pl.when(wid < SEQ)
    def _():
        s = wid
        z0 = pltpu.async_copy(zeros_hbm, buf0, sem0)
        z1 = pltpu.async_copy(zeros_hbm, buf1, sem1)
        pltpu.sync_copy(xt_hbm.at[pl.ds(s * BATCH, BATCH)], col)

        lane = lax.iota(jnp.int32, 16)
        ones16 = jnp.full((16,), 1.0, jnp.float32)
        zeros16 = jnp.zeros((16,), jnp.float32)
        zeroidx16 = jnp.zeros((16,), jnp.int32)

        def bucket(g, _):
            kv = col[pl.ds(g * 16, 16)]
            kc = (kv * KDIV_MAGIC) >> 16
            kcv[pl.ds(g * 16, 16)] = kc
            relv[pl.ds(g * 16, 16)] = kv - kc * KCH
            return 0

        lax.fori_loop(0, GROUPS, bucket, 0, unroll=4)
        z0.wait()
        z1.wait()

        bufs = (buf0, buf1)
        sems = (sem0, sem1)

        def sweep(buf, set_kc, clear_kc):
            def body(g, _):
                kc = kcv[pl.ds(g * 16, 16)]
                rel = relv[pl.ds(g * 16, 16)]
                blane = g * 16 + lane
                if clear_kc is not None:
                    plsc.store_scatter(buf, [zeroidx16, rel, blane],
                                       zeros16, mask=kc == clear_kc)
                plsc.store_scatter(buf, [zeroidx16, rel, blane],
                                   ones16, mask=kc == set_kc)
                return 0

            lax.fori_loop(0, GROUPS, body, 0, unroll=4)

        pending = [None, None]
        for kc in range(NKC):
            b = kc % 2
            if pending[b] is not None:
                pending[b].wait()
            sweep(bufs[b], kc, kc - 2 if kc >= 2 else None)
            dst = out_hbm.at[pl.ds(s, 1), pl.ds(kc * KCH, KCH),
                             pl.ds(0, BATCH)]
            pending[b] = pltpu.async_copy(bufs[b], dst, sems[b])
        for b in (0, 1):
            pending[b].wait()


def kernel(x):
    xt = x.astype(jnp.int32).T.reshape(SEQ * BATCH)
    zeros = jnp.zeros((1, KCH, BATCH), jnp.float32)
    y = _onehot_sc(xt, zeros)
    return jnp.transpose(y, (2, 0, 1))
